# baseline (speedup 1.0000x reference)
import dataclasses
import functools

import jax
import jax.numpy as jnp
from jax import lax
from jax.experimental import pallas as pl
from jax.experimental.pallas import tpu as pltpu
from jax.experimental.pallas import tpu_sc as plsc

_NUM_SUBCORES = 16
_LANES = 16
_D = 64
_HOUT = 8


def _sc_compiler_params():
    cp = pltpu.CompilerParams()
    if "needs_layout_passes" in pltpu.CompilerParams.__dataclass_fields__:
        cp = dataclasses.replace(cp, needs_layout_passes=False)
    return cp


def _round_bf16(x):
    return x.astype(jnp.bfloat16).astype(jnp.float32)


def _round_bf16_bits(x):
    u = plsc.bitcast(x, jnp.int32)
    lsb = lax.bitwise_and(lax.shift_right_logical(u, 16), 1)
    u = lax.bitwise_and(u + 0x7FFF + lsb, jnp.int32(-65536))
    return plsc.bitcast(u, jnp.float32)


def _make_sc_kernel(num_ids: int):
    per_tile = num_ids // _NUM_SUBCORES
    inv_l = 1.0 / num_ids
    mesh = plsc.VectorSubcoreMesh(
        core_axis_name="c", subcore_axis_name="s",
        num_cores=1, num_subcores=_NUM_SUBCORES)

    @functools.partial(
        pl.kernel,
        out_type=(jax.ShapeDtypeStruct((_LANES,), jnp.float32),
                  jax.ShapeDtypeStruct((16, 16, 16), jnp.int32),
                  jax.ShapeDtypeStruct((16, 4, 16), jnp.float32),
                  jax.ShapeDtypeStruct((4, 16), jnp.float32)),
        mesh=mesh,
        scratch_types=[
            pltpu.VMEM((per_tile,), jnp.int32),
            pltpu.VMEM((16, 16), jnp.int32),
            pltpu.VMEM((16, _D), jnp.float32),
            pltpu.VMEM((_LANES,), jnp.int32),
            pltpu.VMEM((16, 16, 16), jnp.int32),
            pltpu.VMEM((4, 16), jnp.float32),
            pltpu.VMEM((16, 4, 16), jnp.float32),
            pltpu.VMEM((4, 16), jnp.float32),
            pltpu.VMEM((_D, 16), jnp.float32),
            pltpu.VMEM((16,), jnp.float32),
            pltpu.VMEM((16,), jnp.float32),
            pltpu.VMEM((4, 16), jnp.float32),
            pltpu.VMEM((_D, 16), jnp.float32),
            pltpu.VMEM((16,), jnp.float32),
            pltpu.VMEM((16,), jnp.float32),
        ],
        compiler_params=_sc_compiler_params(),
    )
    def sc_kernel(ids_hbm, emb_hbm, w1t_hbm, b1_hbm, w2t_hbm, b2_hbm,
                  out_hbm, hstage_hbm, estage_hbm, hhstage_hbm, ids_v, hist2, embt_v, cnt1d, lbuf_i, e_pad,
                  lbuf_e, el_v, w1t_v, b1_v, h1d, hl_v, w2t_v, b2_v,
                  out_v):
        t = lax.axis_index("s")

        pltpu.sync_copy(ids_hbm.at[pl.ds(t * per_tile, per_tile)], ids_v)
        pltpu.sync_copy(emb_hbm.at[pl.ds(t * 16, 16), :], embt_v)

        @pl.when(t < 4)
        def _():
            pltpu.sync_copy(w1t_hbm.at[t], w1t_v)
            pltpu.sync_copy(b1_hbm.at[pl.ds(t * 16, 16)], b1_v)

        @pl.when(t == 0)
        def _():
            pltpu.sync_copy(w2t_hbm, w2t_v)
            pltpu.sync_copy(b2_hbm, b2_v)

        zi = jnp.zeros((_LANES,), jnp.int32)
        zf = jnp.zeros((_LANES,), jnp.float32)

        @pl.loop(0, 16)
        def _(r):
            hist2[r, :] = zi

        @pl.loop(0, per_tile, step=_LANES)
        def _(i):
            idv = ids_v[pl.ds(i, _LANES)]
            cnts, last = plsc.scan_count(idv)
            hi = lax.shift_right_logical(idv, 4)
            lo = lax.bitwise_and(idv, 15)
            plsc.addupdate_scatter(hist2, [hi, lo], cnts, mask=last)

        pltpu.sync_copy(hist2, hstage_hbm.at[t])
        plsc.subcore_barrier()

        pltpu.sync_copy(hstage_hbm, lbuf_i)
        cnt = lbuf_i[0, t, :]
        for k in range(1, 16):
            cnt = cnt + lbuf_i[k, t, :]
        cnt1d[...] = cnt

        def e_body(b, acc):
            e0, e1, e2, e3 = acc
            bc = plsc.load_gather(
                cnt1d, [jnp.full((_LANES,), b, jnp.int32)]).astype(jnp.float32)
            e0 = e0 + bc * embt_v[b, pl.ds(0, 16)]
            e1 = e1 + bc * embt_v[b, pl.ds(16, 16)]
            e2 = e2 + bc * embt_v[b, pl.ds(32, 16)]
            e3 = e3 + bc * embt_v[b, pl.ds(48, 16)]
            return (e0, e1, e2, e3)

        e0, e1, e2, e3 = lax.fori_loop(0, 16, e_body, (zf, zf, zf, zf))
        e_pad[0, :] = e0 * inv_l
        e_pad[1, :] = e1 * inv_l
        e_pad[2, :] = e2 * inv_l
        e_pad[3, :] = e3 * inv_l
        pltpu.sync_copy(e_pad, estage_hbm.at[t])
        plsc.subcore_barrier()

        @pl.when(t < 4)
        def _():
            pltpu.sync_copy(estage_hbm, lbuf_e)
            for r in range(4):
                er = lbuf_e[0, r, :]
                for k in range(1, 16):
                    er = er + lbuf_e[k, r, :]
                el_v[r, :] = er

            def w1_body(d, h):
                bc = plsc.load_gather(
                    el_v, [jnp.full((_LANES,), lax.shift_right_logical(d, 4),
                                    jnp.int32),
                           jnp.full((_LANES,), lax.bitwise_and(d, 15),
                                    jnp.int32)])
                return h + _round_bf16_bits(bc) * w1t_v[d, :]

            h = lax.fori_loop(0, _D, w1_body, zf)
            h1d[...] = jnp.maximum(h + b1_v[...], 0.0)
            pltpu.sync_copy(h1d, hhstage_hbm.at[t])

        plsc.subcore_barrier()

        @pl.when(t == 0)
        def _():
            pltpu.sync_copy(hhstage_hbm, hl_v)

            def w2_body(d, v):
                bc = plsc.load_gather(
                    hl_v, [jnp.full((_LANES,), lax.shift_right_logical(d, 4),
                                    jnp.int32),
                           jnp.full((_LANES,), lax.bitwise_and(d, 15),
                                    jnp.int32)])
                return v + _round_bf16_bits(bc) * w2t_v[d, :]

            v = lax.fori_loop(0, _D, w2_body, zf)
            out_v[...] = v + b2_v[...]
            pltpu.sync_copy(out_v, out_hbm)

    return sc_kernel


def kernel(ids, emb, W1, b1, W2, b2):
    num_ids = ids.shape[0]
    w1t = _round_bf16(W1).T.reshape(_D, 4, _LANES).transpose(1, 0, 2)
    w2t = jnp.zeros((_D, _LANES), jnp.float32).at[:, :_HOUT].set(
        _round_bf16(W2).T)
    b2p = jnp.zeros((_LANES,), jnp.float32).at[:_HOUT].set(b2)
    out = _make_sc_kernel(num_ids)(
        ids.astype(jnp.int32), emb, w1t, b1, w2t, b2p)[0]
    return out[:_HOUT]


# speedup vs baseline: 1.4797x; 1.4797x over previous
import dataclasses
import functools

import jax
import jax.numpy as jnp
from jax import lax
from jax.experimental import pallas as pl
from jax.experimental.pallas import tpu as pltpu
from jax.experimental.pallas import tpu_sc as plsc

_NUM_CORES = 1
_NUM_SUBCORES = 16
_NUM_TILES = _NUM_CORES * _NUM_SUBCORES
_LANES = 16
_BINS = 256


def _sc_compiler_params():
    cp = pltpu.CompilerParams()
    if "needs_layout_passes" in pltpu.CompilerParams.__dataclass_fields__:
        cp = dataclasses.replace(cp, needs_layout_passes=False)
    return cp


def _make_sc_histogram(num_ids: int):
    per_tile = num_ids // _NUM_TILES
    mesh = plsc.VectorSubcoreMesh(
        core_axis_name="c", subcore_axis_name="s",
        num_cores=_NUM_CORES, num_subcores=_NUM_SUBCORES)

    @functools.partial(
        pl.kernel,
        out_type=jax.ShapeDtypeStruct((_NUM_TILES, _BINS), jnp.int32),
        mesh=mesh,
        scratch_types=[
            pltpu.VMEM((per_tile,), jnp.int32),
            pltpu.VMEM((_BINS,), jnp.int32),
        ],
        compiler_params=_sc_compiler_params(),
    )
    def sc_histogram(ids_hbm, out_hbm, ids_v, hist_v):
        wid = lax.axis_index("s") * _NUM_CORES + lax.axis_index("c")
        pltpu.sync_copy(ids_hbm.at[pl.ds(wid * per_tile, per_tile)], ids_v)

        zeros = jnp.zeros((_LANES,), jnp.int32)

        for i in range(0, _BINS, _LANES):
            hist_v[pl.ds(i, _LANES)] = zeros

        @pl.loop(0, per_tile, step=2 * _LANES)
        def _(i):
            for j in range(2):
                ids_vec = ids_v[pl.ds(i + j * _LANES, _LANES)]
                counts, last = plsc.scan_count(ids_vec)
                plsc.addupdate_scatter(hist_v, [ids_vec], counts, mask=last)

        pltpu.sync_copy(hist_v, out_hbm.at[wid])

    return sc_histogram


def _tc_head(parts, emb, W1, b1, W2, b2, inv_l):
    def body(parts_ref, emb_ref, w1_ref, b1_ref, w2_ref, b2_ref, out_ref):
        hi = lax.Precision.HIGHEST
        counts = jnp.sum(parts_ref[...].astype(jnp.float32),
                         axis=0, keepdims=True)
        e = lax.dot_general(
            counts, emb_ref[...], (((1,), (0,)), ((), ())),
            precision=hi, preferred_element_type=jnp.float32) * inv_l
        h = lax.dot_general(
            e, w1_ref[...], (((1,), (1,)), ((), ())),
            preferred_element_type=jnp.float32) + b1_ref[...][None, :]
        h = jnp.maximum(h, 0.0)
        v = lax.dot_general(
            h, w2_ref[...], (((1,), (1,)), ((), ())),
            preferred_element_type=jnp.float32) + b2_ref[...][None, :]
        out_ref[...] = v[0]

    return pl.pallas_call(
        body,
        out_shape=jax.ShapeDtypeStruct((8,), jnp.float32),
    )(parts, emb, W1, b1, W2, b2)


def kernel(ids, emb, W1, b1, W2, b2):
    num_ids = ids.shape[0]
    parts = _make_sc_histogram(num_ids)(ids.astype(jnp.int32))
    return _tc_head(parts, emb, W1, b1, W2, b2, 1.0 / num_ids)


# speedup vs baseline: 1.5109x; 1.0211x over previous
import dataclasses
import functools

import jax
import jax.numpy as jnp
from jax import lax
from jax.experimental import pallas as pl
from jax.experimental.pallas import tpu as pltpu
from jax.experimental.pallas import tpu_sc as plsc

_NUM_CORES = 1
_NUM_SUBCORES = 16
_NUM_TILES = _NUM_CORES * _NUM_SUBCORES
_LANES = 16
_BINS = 256


def _sc_compiler_params():
    cp = pltpu.CompilerParams()
    if "needs_layout_passes" in pltpu.CompilerParams.__dataclass_fields__:
        cp = dataclasses.replace(cp, needs_layout_passes=False)
    return cp


def _make_sc_histogram(num_ids: int):
    per_tile = num_ids // _NUM_TILES
    mesh = plsc.VectorSubcoreMesh(
        core_axis_name="c", subcore_axis_name="s",
        num_cores=_NUM_CORES, num_subcores=_NUM_SUBCORES)

    @functools.partial(
        pl.kernel,
        out_type=jax.ShapeDtypeStruct((_NUM_TILES, _BINS), jnp.int32),
        mesh=mesh,
        scratch_types=[
            pltpu.VMEM((per_tile,), jnp.int32),
            pltpu.VMEM((_BINS,), jnp.int32),
        ],
        compiler_params=_sc_compiler_params(),
    )
    def sc_histogram(ids_hbm, out_hbm, ids_v, hist_v):
        wid = lax.axis_index("s") * _NUM_CORES + lax.axis_index("c")
        pltpu.sync_copy(ids_hbm.at[pl.ds(wid * per_tile, per_tile)], ids_v)

        zeros = jnp.zeros((_LANES,), jnp.int32)

        @pl.loop(0, _BINS, step=_LANES)
        def _(i):
            hist_v[pl.ds(i, _LANES)] = zeros

        @pl.loop(0, per_tile, step=_LANES)
        def _(i):
            ids_vec = ids_v[pl.ds(i, _LANES)]
            counts, last = plsc.scan_count(ids_vec)
            plsc.addupdate_scatter(hist_v, [ids_vec], counts, mask=last)

        pltpu.sync_copy(hist_v, out_hbm.at[wid])

    return sc_histogram


def _tc_head(parts, emb, W1, b1, W2, b2, inv_l):
    def body(parts_ref, emb_ref, w1_ref, b1_ref, w2_ref, b2_ref, out_ref):
        hi = lax.Precision.HIGHEST
        counts = jnp.sum(parts_ref[...].astype(jnp.float32),
                         axis=0, keepdims=True)
        e = lax.dot_general(
            counts, emb_ref[...], (((1,), (0,)), ((), ())),
            precision=hi, preferred_element_type=jnp.float32) * inv_l
        h = lax.dot_general(
            e, w1_ref[...], (((1,), (1,)), ((), ())),
            preferred_element_type=jnp.float32) + b1_ref[...][None, :]
        h = jnp.maximum(h, 0.0)
        v = lax.dot_general(
            h, w2_ref[...], (((1,), (1,)), ((), ())),
            preferred_element_type=jnp.float32) + b2_ref[...][None, :]
        out_ref[...] = v[0]

    return pl.pallas_call(
        body,
        out_shape=jax.ShapeDtypeStruct((8,), jnp.float32),
    )(parts, emb, W1, b1, W2, b2)


def kernel(ids, emb, W1, b1, W2, b2):
    num_ids = ids.shape[0]
    parts = _make_sc_histogram(num_ids)(ids.astype(jnp.int32))
    return _tc_head(parts, emb, W1, b1, W2, b2, 1.0 / num_ids)
